# initial kernel scaffold (unmeasured)
import jax
import jax.numpy as jnp
from jax import lax
from jax.experimental import pallas as pl
from jax.experimental.pallas import tpu as pltpu

N_DEV = 4
N_TOK = 2048
D = 512
H = 1024
E_TOTAL = 32
E_LOCAL = 8
CAP = 51
CAP_PAD = 64
M_PER = E_LOCAL * CAP_PAD


def _ring_moe_kernel(xg, expert_W):

    def body(xg_ref, w_ref, out_ref, comm_ref, send_sems, recv_sems):
        my_pos = lax.axis_index("i")
        left = (my_pos - 1) % N_DEV
        right = (my_pos + 1) % N_DEV

        barrier_sem = pltpu.get_barrier_semaphore()
        for nbr in [left, right]:
            pl.semaphore_signal(
                barrier_sem, inc=1,
                device_id=(nbr,), device_id_type=pl.DeviceIdType.MESH,
            )
        pl.semaphore_wait(barrier_sem, 2)

        for e in range(E_LOCAL):
            comm_ref[0, e * CAP_PAD:(e + 1) * CAP_PAD, :] = jnp.dot(
                xg_ref[e * CAP_PAD:(e + 1) * CAP_PAD, :],
                w_ref[e],
                preferred_element_type=jnp.float32,
            )
        out_ref[pl.ds(my_pos * M_PER, M_PER), :] = comm_ref[0]

        for h in range(N_DEV - 1):
            rdma = pltpu.make_async_remote_copy(
                src_ref=comm_ref.at[h],
                dst_ref=comm_ref.at[h + 1],
                send_sem=send_sems.at[h],
                recv_sem=recv_sems.at[h + 1],
                device_id=(right,),
                device_id_type=pl.DeviceIdType.MESH,
            )
            rdma.start()
            rdma.wait()
            origin = (my_pos - h - 1) % N_DEV
            out_ref[pl.ds(origin * M_PER, M_PER), :] = comm_ref[h + 1]

    return pl.pallas_call(
        body,
        out_shape=jax.ShapeDtypeStruct((N_DEV * M_PER, H), jnp.float32),
        in_specs=[
            pl.BlockSpec(memory_space=pltpu.VMEM),
            pl.BlockSpec(memory_space=pltpu.VMEM),
        ],
        out_specs=pl.BlockSpec(memory_space=pltpu.VMEM),
        scratch_shapes=[
            pltpu.VMEM((N_DEV, M_PER, H), jnp.float32),
            pltpu.SemaphoreType.DMA((N_DEV,)),
            pltpu.SemaphoreType.DMA((N_DEV,)),
        ],
        compiler_params=pltpu.CompilerParams(collective_id=0),
    )(xg, expert_W)


def kernel(x, router_W, route_idx, expert_W):
    del router_W

    e = route_idx[:, 0]
    onehot = (e[:, None] == jnp.arange(E_TOTAL, dtype=e.dtype)[None, :])
    pos = jnp.cumsum(onehot.astype(jnp.int32), axis=0) - 1
    pos_t = jnp.take_along_axis(pos, e[:, None].astype(jnp.int32), axis=1)[:, 0]
    valid = pos_t < CAP
    slot = jnp.where(valid, e.astype(jnp.int32) * CAP_PAD + pos_t, N_TOK)
    token_for_slot = (
        jnp.full((E_TOTAL * CAP_PAD,), N_TOK, jnp.int32)
        .at[slot]
        .set(jnp.arange(N_TOK, dtype=jnp.int32), mode="drop")
    )

    my_pos = lax.axis_index("i")
    my_slots = lax.dynamic_slice(token_for_slot, (my_pos * M_PER,), (M_PER,))
    x_pad = jnp.concatenate([x, jnp.zeros((1, D), x.dtype)], axis=0)
    xg = x_pad[my_slots]

    allres = _ring_moe_kernel(xg, expert_W)

    allres_pad = jnp.concatenate(
        [allres, jnp.zeros((1, H), jnp.float32)], axis=0
    )
    return allres_pad[slot]


# baseline (device time: 67449 ns/iter reference)
import jax
import jax.numpy as jnp
from jax import lax
from jax.experimental import pallas as pl
from jax.experimental.pallas import tpu as pltpu

N_DEV = 4
N_TOK = 2048
D = 512
H = 1024
E_TOTAL = 32
E_LOCAL = 8
CAP = 51
CAP_PAD = 64
M_PER = E_LOCAL * CAP_PAD


def _moe_ring_kernel(x, expert_W, route_idx):
    def body(x_ref, w_ref, e_ref, out_ref, xg_ref, comm_ref,
             send_sems, recv_sems):
        my_pos = lax.axis_index("i")
        left = (my_pos - 1) % N_DEV
        right = (my_pos + 1) % N_DEV

        barrier_sem = pltpu.get_barrier_semaphore()
        for nbr in [left, right]:
            pl.semaphore_signal(
                barrier_sem, inc=1,
                device_id=(nbr,), device_id_type=pl.DeviceIdType.MESH,
            )
        pl.semaphore_wait(barrier_sem, 2)

        iota_e = lax.broadcasted_iota(jnp.int32, (N_TOK, E_TOTAL), 1)
        oh = e_ref[:, :] == iota_e
        tri_r = lax.broadcasted_iota(jnp.int32, (N_TOK, N_TOK), 0)
        tri_c = lax.broadcasted_iota(jnp.int32, (N_TOK, N_TOK), 1)
        tri = (tri_c <= tri_r).astype(jnp.bfloat16)
        pos_inc = jnp.dot(
            tri, oh.astype(jnp.bfloat16), preferred_element_type=jnp.float32
        )
        pos_t = (
            jnp.sum(jnp.where(oh, pos_inc, 0.0), axis=1, keepdims=True)
            .astype(jnp.int32) - 1
        )
        slot_col = jnp.where(
            pos_t < CAP, e_ref[:, :] * CAP_PAD + pos_t, N_TOK
        )

        iota_s = lax.broadcasted_iota(jnp.int32, (N_TOK, M_PER), 1)

        p_local = (slot_col == my_pos * M_PER + iota_s).astype(jnp.float32)
        xg_ref[:, :] = lax.dot_general(
            p_local, x_ref[:, :],
            dimension_numbers=(((0,), (0,)), ((), ())),
            preferred_element_type=jnp.float32,
        )

        for e in range(E_LOCAL):
            comm_ref[0, e * CAP_PAD:(e + 1) * CAP_PAD, :] = jnp.dot(
                xg_ref[e * CAP_PAD:(e + 1) * CAP_PAD, :],
                w_ref[e],
                preferred_element_type=jnp.float32,
            ).astype(jnp.bfloat16)

        def accum(h):
            origin = (my_pos - h) % N_DEV
            p2 = (slot_col == origin * M_PER + iota_s).astype(jnp.bfloat16)
            contrib = jnp.dot(
                p2, comm_ref[h], preferred_element_type=jnp.float32
            )
            if h == 0:
                out_ref[:, :] = contrib
            else:
                out_ref[:, :] = out_ref[:, :] + contrib

        for h in range(N_DEV - 1):
            rdma = pltpu.make_async_remote_copy(
                src_ref=comm_ref.at[h],
                dst_ref=comm_ref.at[h + 1],
                send_sem=send_sems.at[h],
                recv_sem=recv_sems.at[h + 1],
                device_id=(right,),
                device_id_type=pl.DeviceIdType.MESH,
            )
            rdma.start()
            accum(h)
            rdma.wait()
        accum(N_DEV - 1)

    return pl.pallas_call(
        body,
        out_shape=jax.ShapeDtypeStruct((N_TOK, H), jnp.float32),
        in_specs=[
            pl.BlockSpec(memory_space=pltpu.VMEM),
            pl.BlockSpec(memory_space=pltpu.VMEM),
            pl.BlockSpec(memory_space=pltpu.VMEM),
        ],
        out_specs=pl.BlockSpec(memory_space=pltpu.VMEM),
        scratch_shapes=[
            pltpu.VMEM((M_PER, D), jnp.float32),
            pltpu.VMEM((N_DEV, M_PER, H), jnp.bfloat16),
            pltpu.SemaphoreType.DMA((N_DEV,)),
            pltpu.SemaphoreType.DMA((N_DEV,)),
        ],
        compiler_params=pltpu.CompilerParams(collective_id=0),
    )(x, expert_W, route_idx)


def kernel(x, router_W, route_idx, expert_W):
    del router_W
    return _moe_ring_kernel(x, expert_W, route_idx.astype(jnp.int32))


# device time: 37406 ns/iter; 1.8032x vs baseline; 1.8032x over previous
import jax
import jax.numpy as jnp
from jax import lax
from jax.experimental import pallas as pl
from jax.experimental.pallas import tpu as pltpu

N_DEV = 4
N_TOK = 2048
D = 512
H = 1024
E_TOTAL = 32
E_LOCAL = 8
CAP = 51
CAP_PAD = 64
M_PER = E_LOCAL * CAP_PAD
LOG_TOK = 11


def _moe_ring_kernel(x, expert_W, route_idx):
    def body(x_hbm, w_hbm, e_ref, out_ref, x_ref, w_ref,
             xg_ref, comm_ref, p_scr, load_sems, send_sems, recv_sems):
        my_pos = lax.axis_index("i")
        left = (my_pos - 1) % N_DEV
        right = (my_pos + 1) % N_DEV

        x_load = pltpu.make_async_copy(x_hbm, x_ref, load_sems.at[0])
        w_load = pltpu.make_async_copy(w_hbm, w_ref, load_sems.at[1])
        x_load.start()
        w_load.start()

        barrier_sem = pltpu.get_barrier_semaphore()
        for nbr in [left, right]:
            pl.semaphore_signal(
                barrier_sem, inc=1,
                device_id=(nbr,), device_id_type=pl.DeviceIdType.MESH,
            )

        iota_e = lax.broadcasted_iota(jnp.int32, (N_TOK, E_TOTAL), 1)
        oh = e_ref[:, :] == iota_e
        row = lax.broadcasted_iota(jnp.int32, (N_TOK, E_TOTAL), 0)
        s = oh.astype(jnp.int32)
        for k in range(LOG_TOK):
            sh = 1 << k
            s = s + jnp.where(row >= sh, pltpu.roll(s, sh, 0), 0)
        pos_t = jnp.sum(jnp.where(oh, s, 0), axis=1, keepdims=True) - 1
        slot_col = jnp.where(
            pos_t < CAP, e_ref[:, :] * CAP_PAD + pos_t, N_TOK
        )

        iota_s = lax.broadcasted_iota(jnp.int32, (N_TOK, M_PER), 1)

        def onehot_block(origin, dtype):
            return (slot_col - origin * M_PER == iota_s).astype(dtype)

        p_mine = onehot_block(my_pos, jnp.bfloat16)
        x_load.wait()
        xg_ref[:, :] = lax.dot_general(
            p_mine, x_ref[:, :].astype(jnp.bfloat16),
            dimension_numbers=(((0,), (0,)), ((), ())),
            preferred_element_type=jnp.float32,
        )

        w_load.wait()
        for e in range(E_LOCAL):
            comm_ref[0, e * CAP_PAD:(e + 1) * CAP_PAD, :] = jnp.dot(
                xg_ref[e * CAP_PAD:(e + 1) * CAP_PAD, :],
                w_ref[e],
                preferred_element_type=jnp.float32,
            ).astype(jnp.bfloat16)

        def rdma(k, src, dst, dev):
            return pltpu.make_async_remote_copy(
                src_ref=src, dst_ref=dst,
                send_sem=send_sems.at[k], recv_sem=recv_sems.at[k],
                device_id=(dev,), device_id_type=pl.DeviceIdType.MESH,
            )

        A = slice(0, H // 2)
        B = slice(H // 2, H)
        r0a = rdma(0, comm_ref.at[0, :, A], comm_ref.at[1, :, A], right)
        r0b = rdma(1, comm_ref.at[0, :, B], comm_ref.at[1, :, B], right)
        r1a = rdma(2, comm_ref.at[0, :, B], comm_ref.at[2, :, B], left)
        r1b = rdma(3, comm_ref.at[0, :, A], comm_ref.at[2, :, A], left)
        r2 = rdma(4, comm_ref.at[1, :, A], comm_ref.at[3, :, A], right)
        r3 = rdma(5, comm_ref.at[2, :, B], comm_ref.at[3, :, B], left)

        pl.semaphore_wait(barrier_sem, 2)
        r0a.start()
        r1a.start()
        r0b.start()
        r1b.start()

        out_ref[:, :] = jnp.dot(
            p_mine, comm_ref[0], preferred_element_type=jnp.float32
        )
        p_scr[0, :, :] = onehot_block(left, jnp.bfloat16)
        p_scr[1, :, :] = onehot_block(right, jnp.bfloat16)
        p_scr[2, :, :] = onehot_block((my_pos + 2) % N_DEV, jnp.bfloat16)

        r0a.wait_recv()
        r2.start()
        r1a.wait_recv()
        r3.start()

        r0b.wait_recv()
        r1b.wait_recv()

        def accum(k, chunk):
            out_ref[:, :] = out_ref[:, :] + jnp.dot(
                p_scr[k], chunk, preferred_element_type=jnp.float32
            )

        accum(0, comm_ref[1])
        accum(1, comm_ref[2])

        r2.wait_recv()
        r3.wait_recv()
        accum(2, comm_ref[3])

        for r in (r0a, r0b, r1a, r1b, r2, r3):
            r.wait_send()

    return pl.pallas_call(
        body,
        out_shape=jax.ShapeDtypeStruct((N_TOK, H), jnp.float32),
        in_specs=[
            pl.BlockSpec(memory_space=pl.ANY),
            pl.BlockSpec(memory_space=pl.ANY),
            pl.BlockSpec(memory_space=pltpu.VMEM),
        ],
        out_specs=pl.BlockSpec(memory_space=pltpu.VMEM),
        scratch_shapes=[
            pltpu.VMEM((N_TOK, D), jnp.float32),
            pltpu.VMEM((E_LOCAL, D, H), jnp.float32),
            pltpu.VMEM((M_PER, D), jnp.float32),
            pltpu.VMEM((N_DEV, M_PER, H), jnp.bfloat16),
            pltpu.VMEM((3, N_TOK, M_PER), jnp.bfloat16),
            pltpu.SemaphoreType.DMA((2,)),
            pltpu.SemaphoreType.DMA((6,)),
            pltpu.SemaphoreType.DMA((6,)),
        ],
        compiler_params=pltpu.CompilerParams(collective_id=0),
    )(x, expert_W, route_idx)


def kernel(x, router_W, route_idx, expert_W):
    del router_W
    return _moe_ring_kernel(x, expert_W, route_idx.astype(jnp.int32))
